# baseline (speedup 1.0000x reference)
import jax
import jax.numpy as jnp
from jax.experimental import pallas as pl
from jax.experimental.pallas import tpu as pltpu

_Q = 128
_A = 64
_B = 16
_BM = 0
_L = 512
_K = 4


def _exp2i(biased):
    return jax.lax.bitcast_convert_type(
        jax.lax.shift_left(254 - biased, 23), jnp.float32)


def _exponent(s):
    return jax.lax.shift_right_logical(
        jax.lax.bitcast_convert_type(s, jnp.int32), 23) & 0xFF


def _fwd_kernel(x_ref, tl_ref, f_ref, out_ref, tp_ref, t2_ref):
    logits = tl_ref[...]
    m = jnp.max(logits, axis=-1, keepdims=True)
    e = jnp.exp(logits - m)
    p = e / jnp.sum(e, axis=-1, keepdims=True)
    tp_ref[0:_A] = p.astype(jnp.bfloat16)
    t2_ref[0:_A] = jnp.transpose(p, (0, 2, 1)).astype(jnp.bfloat16)
    r = jax.lax.broadcasted_iota(jnp.int32, (_Q, _Q), 0)
    c = jax.lax.broadcasted_iota(jnp.int32, (_Q, _Q), 1)
    eye = jnp.where(r == c, 1.0, 0.0).astype(jnp.bfloat16)
    tp_ref[_A] = eye
    t2_ref[_A] = eye

    fl = f_ref[...]
    fe = jnp.exp(fl - jnp.max(fl))
    fprob = fe / jnp.sum(fe)

    lane = jax.lax.broadcasted_iota(jnp.int32, (1, _Q), 1)
    row0 = jnp.where(lane == 0, 1.0, 0.0)
    rows0 = tuple(row0.astype(jnp.bfloat16) for _ in range(_BM)) + tuple(
        row0.astype(jnp.float32) for _ in range(_B - _BM))
    esum_init = tuple(jnp.zeros((1, 1), jnp.int32) for _ in range(_B))

    def body(i, carry):
        rows, esums = carry
        rows = list(rows)
        for k in range(_K):
            t = i * _K + k
            for b in range(_BM):
                sym = x_ref[b, t]
                tb = tp_ref[sym]
                rows[b] = jax.lax.dot_general(
                    rows[b], tb, (((1,), (0,)), ((), ())),
                    preferred_element_type=jnp.float32).astype(jnp.bfloat16)
            for b in range(_BM, _B):
                sym = x_ref[b, t]
                if k % 2 == 0:
                    prod = t2_ref[sym] * rows[b]
                    rows[b] = jnp.sum(prod, axis=1, keepdims=True)
                else:
                    prod = tp_ref[sym] * rows[b]
                    rows[b] = jnp.sum(prod, axis=0, keepdims=True)
        new_rows, new_esums = [], []
        for b in range(_B):
            s = jnp.sum(rows[b].astype(jnp.float32),
                        axis=None, keepdims=True)
            biased = _exponent(s)
            scale = _exp2i(biased)
            new_rows.append((rows[b].astype(jnp.float32)
                             * scale).astype(rows[b].dtype))
            new_esums.append(esums[b] + (biased - 127))
        return tuple(new_rows), tuple(new_esums)

    rows, esums = jax.lax.fori_loop(0, _L // _K, body, (rows0, esum_init))
    ln2 = 0.6931471805599453
    for b in range(_B):
        rs = jnp.sum(rows[b].astype(jnp.float32).reshape(1, _Q) * fprob,
                     axis=1, keepdims=True)
        tot = jnp.log(rs) + esums[b].astype(jnp.float32) * ln2
        out_ref[b:b + 1, :] = jnp.broadcast_to(tot, (1, _Q))


def kernel(x, lengths, T_logits, f_logits):
    tl = jnp.transpose(T_logits, (1, 0, 2))
    fl = f_logits.reshape(1, _Q)
    t_idx = jnp.arange(_L, dtype=jnp.int32)[None, :]
    xm = jnp.where(t_idx < lengths.astype(jnp.int32)[:, None],
                   x.astype(jnp.int32), _A)
    out = pl.pallas_call(
        _fwd_kernel,
        out_shape=jax.ShapeDtypeStruct((_B, _Q), jnp.float32),
        in_specs=[
            pl.BlockSpec(memory_space=pltpu.SMEM),
            pl.BlockSpec(memory_space=pltpu.VMEM),
            pl.BlockSpec(memory_space=pltpu.VMEM),
        ],
        out_specs=pl.BlockSpec(memory_space=pltpu.VMEM),
        scratch_shapes=[pltpu.VMEM((_A + 1, _Q, _Q), jnp.bfloat16),
                        pltpu.VMEM((_A + 1, _Q, _Q), jnp.bfloat16)],
    )(xm, tl, fl)
    return out[:, 0]


# speedup vs baseline: 1.1464x; 1.1464x over previous
import jax
import jax.numpy as jnp
from jax.experimental import pallas as pl
from jax.experimental.pallas import tpu as pltpu

_Q = 128
_A = 64
_B = 16
_BM = 4
_L = 512
_K = 4


def _exp2i(biased):
    return jax.lax.bitcast_convert_type(
        jax.lax.shift_left(254 - biased, 23), jnp.float32)


def _exponent(s):
    return jax.lax.shift_right_logical(
        jax.lax.bitcast_convert_type(s, jnp.int32), 23) & 0xFF


def _fwd_kernel(x_ref, tl_ref, f_ref, out_ref, tp_ref, t2_ref):
    logits = tl_ref[...]
    m = jnp.max(logits, axis=-1, keepdims=True)
    e = jnp.exp(logits - m)
    p = e / jnp.sum(e, axis=-1, keepdims=True)
    tp_ref[0:_A] = p.astype(jnp.bfloat16)
    t2_ref[0:_A] = jnp.transpose(p, (0, 2, 1)).astype(jnp.bfloat16)
    r = jax.lax.broadcasted_iota(jnp.int32, (_Q, _Q), 0)
    c = jax.lax.broadcasted_iota(jnp.int32, (_Q, _Q), 1)
    eye = jnp.where(r == c, 1.0, 0.0).astype(jnp.bfloat16)
    tp_ref[_A] = eye
    t2_ref[_A] = eye

    fl = f_ref[...]
    fe = jnp.exp(fl - jnp.max(fl))
    fprob = fe / jnp.sum(fe)

    lane = jax.lax.broadcasted_iota(jnp.int32, (1, _Q), 1)
    row0 = jnp.where(lane == 0, 1.0, 0.0)
    rows0 = tuple(row0.astype(jnp.bfloat16) for _ in range(_BM)) + tuple(
        row0.astype(jnp.float32) for _ in range(_B - _BM))
    esum_init = tuple(jnp.zeros((1, 1), jnp.int32) for _ in range(_B))

    def body(i, carry):
        rows, esums = carry
        rows = list(rows)
        for k in range(_K):
            t = i * _K + k
            for b in range(_BM):
                sym = x_ref[b, t]
                tb = tp_ref[sym]
                rows[b] = jax.lax.dot_general(
                    rows[b], tb, (((1,), (0,)), ((), ())),
                    preferred_element_type=jnp.float32).astype(jnp.bfloat16)
            for b in range(_BM, _B):
                sym = x_ref[b, t]
                if k % 2 == 0:
                    prod = t2_ref[sym] * rows[b]
                    rows[b] = jnp.sum(prod, axis=1, keepdims=True)
                else:
                    prod = tp_ref[sym] * rows[b]
                    rows[b] = jnp.sum(prod, axis=0, keepdims=True)
        new_rows, new_esums = [], []
        for b in range(_B):
            s = jnp.sum(rows[b].astype(jnp.float32),
                        axis=None, keepdims=True)
            biased = _exponent(s)
            scale = _exp2i(biased)
            new_rows.append((rows[b].astype(jnp.float32)
                             * scale).astype(rows[b].dtype))
            new_esums.append(esums[b] + (biased - 127))
        return tuple(new_rows), tuple(new_esums)

    rows, esums = jax.lax.fori_loop(0, _L // _K, body, (rows0, esum_init))
    ln2 = 0.6931471805599453
    for b in range(_B):
        rs = jnp.sum(rows[b].astype(jnp.float32).reshape(1, _Q) * fprob,
                     axis=1, keepdims=True)
        tot = jnp.log(rs) + esums[b].astype(jnp.float32) * ln2
        out_ref[b:b + 1, :] = jnp.broadcast_to(tot, (1, _Q))


def kernel(x, lengths, T_logits, f_logits):
    tl = jnp.transpose(T_logits, (1, 0, 2))
    fl = f_logits.reshape(1, _Q)
    t_idx = jnp.arange(_L, dtype=jnp.int32)[None, :]
    xm = jnp.where(t_idx < lengths.astype(jnp.int32)[:, None],
                   x.astype(jnp.int32), _A)
    out = pl.pallas_call(
        _fwd_kernel,
        out_shape=jax.ShapeDtypeStruct((_B, _Q), jnp.float32),
        in_specs=[
            pl.BlockSpec(memory_space=pltpu.SMEM),
            pl.BlockSpec(memory_space=pltpu.VMEM),
            pl.BlockSpec(memory_space=pltpu.VMEM),
        ],
        out_specs=pl.BlockSpec(memory_space=pltpu.VMEM),
        scratch_shapes=[pltpu.VMEM((_A + 1, _Q, _Q), jnp.bfloat16),
                        pltpu.VMEM((_A + 1, _Q, _Q), jnp.bfloat16)],
    )(xm, tl, fl)
    return out[:, 0]
